# baseline (speedup 1.0000x reference)
import functools
import math
import jax
import jax.numpy as jnp
from jax import lax
from jax.experimental import pallas as pl
from jax.experimental.pallas import tpu as pltpu
from jax.experimental.pallas import tpu_sc as plsc

T, N, E, H = 4, 10000, 320000, 128
INV_SQRT_D = 1.0 / math.sqrt(128.0)

NC, NS, LANES = 2, 16, 16
NW = NC * NS
EPW = E // NW
C = 80
NCHUNK = EPW // C
NPAD = 10240
RPT = NPAD // NS

_mesh = plsc.VectorSubcoreMesh(
    core_axis_name="c", subcore_axis_name="s", num_cores=NC, num_subcores=NS)


def _gru_body(x_ref, wih_ref, whh_ref, bih_ref, bhh_ref, out_ref, h_ref):
    t = pl.program_id(0)

    @pl.when(t == 0)
    def _():
        h_ref[...] = jnp.zeros_like(h_ref)

    x = x_ref[0]
    h = h_ref[...]
    gi = lax.dot_general(x, wih_ref[...], (((1,), (1,)), ((), ())),
                         preferred_element_type=jnp.float32) + bih_ref[...]
    gh = lax.dot_general(h, whh_ref[...], (((1,), (1,)), ((), ())),
                         preferred_element_type=jnp.float32) + bhh_ref[...]
    r = jax.nn.sigmoid(gi[:, :H] + gh[:, :H])
    z = jax.nn.sigmoid(gi[:, H:2 * H] + gh[:, H:2 * H])
    n = jnp.tanh(gi[:, 2 * H:] + r * gh[:, 2 * H:])
    h = (1.0 - z) * n + z * h
    h_ref[...] = h
    out_ref[0] = h


def _gru_forward(x_seq, gru):
    bih = gru['b_ih'].reshape(1, 3 * H)
    bhh = gru['b_hh'].reshape(1, 3 * H)
    return pl.pallas_call(
        _gru_body,
        grid=(T,),
        in_specs=[
            pl.BlockSpec((1, N, H), lambda t: (t, 0, 0)),
            pl.BlockSpec((3 * H, H), lambda t: (0, 0)),
            pl.BlockSpec((3 * H, H), lambda t: (0, 0)),
            pl.BlockSpec((1, 3 * H), lambda t: (0, 0)),
            pl.BlockSpec((1, 3 * H), lambda t: (0, 0)),
        ],
        out_specs=pl.BlockSpec((1, N, H), lambda t: (t, 0, 0)),
        out_shape=jax.ShapeDtypeStruct((T, N, H), jnp.float32),
        scratch_shapes=[pltpu.VMEM((N, H), jnp.float32)],
    )(x_seq, gru['W_ih'], gru['W_hh'], bih, bhh)


def _proj_body(h_ref, w_ref, b_ref, q_ref, kv_ref, s_ref):
    y = jnp.dot(h_ref[...], w_ref[...],
                preferred_element_type=jnp.float32) + b_ref[...]
    q_ref[...] = y[:, :H]
    kv_ref[...] = y[:, H:3 * H]
    s_ref[...] = y[:, 3 * H:]


def _projections(h, p):
    w = jnp.concatenate([p['Wq'] * INV_SQRT_D, p['Wk'], p['Wv'], p['Wskip']], axis=1)
    b = jnp.concatenate([p['bq'] * INV_SQRT_D, p['bk'], p['bv'], p['bskip']]).reshape(1, 4 * H)
    shp = jax.ShapeDtypeStruct((N, H), jnp.float32)
    shp2 = jax.ShapeDtypeStruct((N, 2 * H), jnp.float32)
    return pl.pallas_call(
        _proj_body,
        out_shape=(shp, shp2, shp),
    )(h, w, b)


def _edge_body(q_hbm, kv_hbm, src_hbm, dst_hbm, ea_hbm, we_hbm,
               out_hbm, den_hbm,
               srcb, dstb, sdst, eab, qb, kvb, ob, exb, web, acc_sh, den_sh,
               sem1, sem2, ss):
    cid = lax.axis_index("c")
    sid = lax.axis_index("s")
    wid = cid * NS + sid
    base_edge = wid * EPW

    pltpu.sync_copy(we_hbm, web)

    z16 = jnp.zeros((16,), jnp.float32)

    def zero_ob_body(i, carry):
        ob[i // 8, pl.ds((i % 8) * 16, 16)] = z16
        return carry

    lax.fori_loop(0, C * 8, zero_ob_body, 0)

    def zero_ex_body(i, carry):
        exb[pl.ds(i * 16, 16)] = z16
        return carry

    lax.fori_loop(0, C // 16, zero_ex_body, 0)

    for i in range(RPT // C):
        pltpu.sync_copy(ob, acc_sh.at[pl.ds(sid * RPT + i * C, C)])
        pltpu.sync_copy(exb, den_sh.at[pl.ds(sid * RPT + i * C, C)])
    plsc.subcore_barrier()

    wevs = [web[pl.ds(16 * j, 16)] for j in range(8)]
    iota16 = lax.iota(jnp.int32, 16)

    def chunk_body(ci, carry):
        bo = base_edge + ci * C
        pltpu.sync_copy(src_hbm.at[pl.ds(bo, C)], srcb)
        pltpu.sync_copy(dst_hbm.at[pl.ds(bo, C)], dstb)
        pltpu.sync_copy(ea_hbm.at[pl.ds(bo, C)], eab)
        cp1 = pltpu.async_copy(q_hbm.at[dstb], qb, sem1)
        cp2 = pltpu.async_copy(kv_hbm.at[srcb], kvb, sem2)
        cp1.wait()
        cp2.wait()

        def group_body(g, carry2):
            sl16 = pl.ds(g * LANES, LANES)
            ea16 = eab[sl16]
            sdst[sl16] = dstb[sl16]
            exg = jnp.zeros((16,), jnp.float32)
            for l in range(LANES):
                e = g * LANES + l
                eae = ea16[l]
                acc = jnp.zeros((16,), jnp.float32)
                for j in range(8):
                    sl = pl.ds(16 * j, 16)
                    acc = acc + qb[e, sl] * (kvb[e, sl] + eae * wevs[j])
                alpha = jnp.sum(acc)
                exv = jnp.exp(jnp.full((16,), alpha, jnp.float32))
                for j in range(8):
                    sl = pl.ds(16 * j, 16)
                    vsl = pl.ds(H + 16 * j, 16)
                    ob[e, sl] = exv * (kvb[e, vsl] + eae * wevs[j])
                exg = exg + exv * jnp.where(iota16 == l, 1.0, 0.0)
            exb[sl16] = exg
            return carry2

        lax.fori_loop(0, C // LANES, group_body, 0)
        pltpu.sync_copy(ob, acc_sh.at[sdst], add=True)
        pltpu.sync_copy(exb, den_sh.at[sdst], add=True)
        return carry

    lax.fori_loop(0, NCHUNK, chunk_body, 0)
    plsc.subcore_barrier()

    pltpu.sync_copy(acc_sh.at[pl.ds(sid * RPT, RPT)],
                    out_hbm.at[cid].at[pl.ds(sid * RPT, RPT)])
    pltpu.sync_copy(den_sh.at[pl.ds(sid * RPT, RPT)],
                    den_hbm.at[cid].at[pl.ds(sid * RPT, RPT)])


_edge_call = functools.partial(
    pl.kernel,
    out_type=(jax.ShapeDtypeStruct((NC, NPAD, H), jnp.float32),
              jax.ShapeDtypeStruct((NC, NPAD), jnp.float32)),
    mesh=_mesh,
    compiler_params=pltpu.CompilerParams(needs_layout_passes=False,
                                         use_tc_tiling_on_sc=False),
    scratch_types=[
        pltpu.VMEM((C,), jnp.int32),
        pltpu.VMEM((C,), jnp.int32),
        pltpu.VMEM((C,), jnp.int32),
        pltpu.VMEM((C,), jnp.float32),
        pltpu.VMEM((C, H), jnp.float32),
        pltpu.VMEM((C, 2 * H), jnp.float32),
        pltpu.VMEM((C, H), jnp.float32),
        pltpu.VMEM((C,), jnp.float32),
        pltpu.VMEM((H,), jnp.float32),
        pltpu.VMEM_SHARED((NPAD, H), jnp.float32),
        pltpu.VMEM_SHARED((NPAD,), jnp.float32),
        pltpu.SemaphoreType.DMA,
        pltpu.SemaphoreType.DMA,
        pltpu.SemaphoreType.DMA,
    ],
)(_edge_body)


def _edge1_body(q_hbm, k_hbm, v_hbm, src_hbm, dst_hbm, ea_hbm, we_hbm,
                num_hbm, den_hbm, srcb, dstb, eab, qbuf, kbuf, vbuf,
                numb, exb, web, num_sh, den_sh):
    cid = lax.axis_index("c")
    sid = lax.axis_index("s")
    wid = cid * NS + sid

    pltpu.sync_copy(we_hbm, web)
    pltpu.sync_copy(q_hbm, qbuf)
    pltpu.sync_copy(k_hbm, kbuf)
    pltpu.sync_copy(v_hbm, vbuf)

    z16 = jnp.zeros((16,), jnp.float32)

    def zero_small(i, carry):
        numb[pl.ds(i * 16, 16)] = z16
        exb[pl.ds(i * 16, 16)] = z16
        return carry

    lax.fori_loop(0, C // 16, zero_small, 0)
    for i in range(RPT // C):
        pltpu.sync_copy(numb, num_sh.at[pl.ds(sid * RPT + i * C, C)])
        pltpu.sync_copy(exb, den_sh.at[pl.ds(sid * RPT + i * C, C)])
    plsc.subcore_barrier()

    base_edge = wid * EPW
    wev = web[...]

    def chunk_body(ci, carry):
        b = base_edge + ci * C
        pltpu.sync_copy(src_hbm.at[pl.ds(b, C)], srcb)
        pltpu.sync_copy(dst_hbm.at[pl.ds(b, C)], dstb)
        pltpu.sync_copy(ea_hbm.at[pl.ds(b, C)], eab)

        def group_body(g, carry2):
            sl = pl.ds(g * LANES, LANES)
            srcv = srcb[sl]
            dstv = dstb[sl]
            eav = eab[sl]
            k16 = plsc.load_gather(kbuf, [srcv])
            q16 = plsc.load_gather(qbuf, [dstv])
            v16 = plsc.load_gather(vbuf, [srcv])
            ew = eav * wev
            ex = jnp.exp(q16 * (k16 + ew))
            numb[sl] = ex * (v16 + ew)
            exb[sl] = ex
            return carry2

        lax.fori_loop(0, C // LANES, group_body, 0)
        pltpu.sync_copy(numb, num_sh.at[dstb], add=True)
        pltpu.sync_copy(exb, den_sh.at[dstb], add=True)
        return carry

    lax.fori_loop(0, NCHUNK, chunk_body, 0)
    plsc.subcore_barrier()

    pltpu.sync_copy(num_sh.at[pl.ds(sid * RPT, RPT)],
                    num_hbm.at[cid].at[pl.ds(sid * RPT, RPT)])
    pltpu.sync_copy(den_sh.at[pl.ds(sid * RPT, RPT)],
                    den_hbm.at[cid].at[pl.ds(sid * RPT, RPT)])


_edge1_call = functools.partial(
    pl.kernel,
    out_type=(jax.ShapeDtypeStruct((NC, NPAD), jnp.float32),
              jax.ShapeDtypeStruct((NC, NPAD), jnp.float32)),
    mesh=_mesh,
    compiler_params=pltpu.CompilerParams(needs_layout_passes=False,
                                         use_tc_tiling_on_sc=False),
    scratch_types=[
        pltpu.VMEM((C,), jnp.int32),
        pltpu.VMEM((C,), jnp.int32),
        pltpu.VMEM((C,), jnp.float32),
        pltpu.VMEM((NPAD,), jnp.float32),
        pltpu.VMEM((NPAD,), jnp.float32),
        pltpu.VMEM((NPAD,), jnp.float32),
        pltpu.VMEM((C,), jnp.float32),
        pltpu.VMEM((C,), jnp.float32),
        pltpu.VMEM((16,), jnp.float32),
        pltpu.VMEM_SHARED((NPAD,), jnp.float32),
        pltpu.VMEM_SHARED((NPAD,), jnp.float32),
    ],
)(_edge1_body)


def _combine_body(acc_ref, den_ref, skip_ref, out_ref):
    num = (acc_ref[0] + acc_ref[1])[:N]
    den = (den_ref[0] + den_ref[1])[:N]
    h = num / (den[:, None] + 1e-16) + skip_ref[...]
    out_ref[...] = jnp.where(h >= 0.0, h, 0.01 * h)


def _combine(acc, den, skip):
    return pl.pallas_call(
        _combine_body,
        out_shape=jax.ShapeDtypeStruct((N, H), jnp.float32),
    )(acc, den, skip)


def _mean_proj_body(a_ref, b_ref, c_ref, d_ref, w_ref, bias_ref, y_ref):
    hm = 0.25 * (a_ref[...] + b_ref[...] + c_ref[...] + d_ref[...])
    y_ref[...] = jnp.dot(hm, w_ref[...],
                         preferred_element_type=jnp.float32) + bias_ref[...]


def _final_body(num_ref, den_ref, skip_ref, out_ref):
    num = (num_ref[0] + num_ref[1])[:N]
    den = (den_ref[0] + den_ref[1])[:N]
    out_ref[...] = (num / (den + 1e-16))[:, None] + skip_ref[...]


def kernel(x_seq, edge_index_seq, edge_attr_seq, gru, conv1, conv2, conv_out):
    gru_out = _gru_forward(x_seq, gru)
    edge_index_seq = edge_index_seq.astype(jnp.int32)

    h_list = []
    for t in range(T):
        h_t = gru_out[t]
        srcf = edge_index_seq[t, 0]
        dstf = edge_index_seq[t, 1]
        eaf = edge_attr_seq[t, :, 0]
        for p in (conv1, conv2):
            q, kv, skip = _projections(h_t, p)
            acc, den = _edge_call(q, kv, srcf, dstf, eaf, p['We'][0])
            h_t = _combine(acc, den, skip)
        h_list.append(h_t)

    p = conv_out
    w4 = jnp.concatenate([p['Wq'], p['Wk'], p['Wv'], p['Wskip']], axis=1)
    w4 = jnp.pad(w4, ((0, 0), (0, 4)))
    b4 = jnp.pad(jnp.concatenate([p['bq'], p['bk'], p['bv'], p['bskip']]),
                 (0, 4)).reshape(1, 8)
    y = pl.pallas_call(
        _mean_proj_body,
        out_shape=jax.ShapeDtypeStruct((N, 8), jnp.float32),
    )(*h_list, w4, b4)

    src = edge_index_seq[T - 1, 0]
    dst = edge_index_seq[T - 1, 1]
    ea = edge_attr_seq[T - 1, :, 0]
    qv = jnp.pad(y[:, 0], (0, NPAD - N))
    kv = jnp.pad(y[:, 1], (0, NPAD - N))
    vv = jnp.pad(y[:, 2], (0, NPAD - N))
    skip = y[:, 3:4]
    we16 = jnp.full((16,), p['We'][0, 0], jnp.float32)
    num, den = _edge1_call(qv, kv, vv, src, dst, ea, we16)
    out = pl.pallas_call(
        _final_body,
        out_shape=jax.ShapeDtypeStruct((N, 1), jnp.float32),
    )(num, den, skip)
    return out


# speedup vs baseline: 1.7998x; 1.7998x over previous
import functools
import math
import jax
import jax.numpy as jnp
from jax import lax
from jax.experimental import pallas as pl
from jax.experimental.pallas import tpu as pltpu
from jax.experimental.pallas import tpu_sc as plsc

T, N, E, H = 4, 10000, 320000, 128
INV_SQRT_D = 1.0 / math.sqrt(128.0)

NC, NS, LANES = 2, 16, 16
NW = NC * NS
EPW = E // NW
C = 80
NCHUNK = EPW // C
NPAD = 10240
RPT = NPAD // NS

_mesh = plsc.VectorSubcoreMesh(
    core_axis_name="c", subcore_axis_name="s", num_cores=NC, num_subcores=NS)


def _gru_body(x_ref, wih_ref, whh_ref, bih_ref, bhh_ref, out_ref, h_ref):
    t = pl.program_id(0)

    @pl.when(t == 0)
    def _():
        h_ref[...] = jnp.zeros_like(h_ref)

    x = x_ref[0]
    h = h_ref[...]
    gi = lax.dot_general(x, wih_ref[...], (((1,), (1,)), ((), ())),
                         preferred_element_type=jnp.float32) + bih_ref[...]
    gh = lax.dot_general(h, whh_ref[...], (((1,), (1,)), ((), ())),
                         preferred_element_type=jnp.float32) + bhh_ref[...]
    r = jax.nn.sigmoid(gi[:, :H] + gh[:, :H])
    z = jax.nn.sigmoid(gi[:, H:2 * H] + gh[:, H:2 * H])
    n = jnp.tanh(gi[:, 2 * H:] + r * gh[:, 2 * H:])
    h = (1.0 - z) * n + z * h
    h_ref[...] = h
    out_ref[0] = h


def _gru_forward(x_seq, gru):
    bih = gru['b_ih'].reshape(1, 3 * H)
    bhh = gru['b_hh'].reshape(1, 3 * H)
    return pl.pallas_call(
        _gru_body,
        grid=(T,),
        in_specs=[
            pl.BlockSpec((1, N, H), lambda t: (t, 0, 0)),
            pl.BlockSpec((3 * H, H), lambda t: (0, 0)),
            pl.BlockSpec((3 * H, H), lambda t: (0, 0)),
            pl.BlockSpec((1, 3 * H), lambda t: (0, 0)),
            pl.BlockSpec((1, 3 * H), lambda t: (0, 0)),
        ],
        out_specs=pl.BlockSpec((1, N, H), lambda t: (t, 0, 0)),
        out_shape=jax.ShapeDtypeStruct((T, N, H), jnp.float32),
        scratch_shapes=[pltpu.VMEM((N, H), jnp.float32)],
    )(x_seq, gru['W_ih'], gru['W_hh'], bih, bhh)


def _proj_body(h_ref, w_ref, b_ref, q_ref, k_ref, v_ref, s_ref):
    y = jnp.dot(h_ref[...], w_ref[...],
                preferred_element_type=jnp.float32) + b_ref[...]
    q_ref[...] = y[:, :H]
    k_ref[...] = y[:, H:2 * H]
    v_ref[...] = y[:, 2 * H:3 * H]
    s_ref[...] = y[:, 3 * H:]


def _projections(h, p):
    w = jnp.concatenate([p['Wq'] * INV_SQRT_D, p['Wk'], p['Wv'], p['Wskip']], axis=1)
    b = jnp.concatenate([p['bq'] * INV_SQRT_D, p['bk'], p['bv'], p['bskip']]).reshape(1, 4 * H)
    shp = jax.ShapeDtypeStruct((N, H), jnp.float32)
    return pl.pallas_call(
        _proj_body,
        out_shape=(shp, shp, shp, shp),
    )(h, w, b)


def _edge_body(q_hbm, k_hbm, v_hbm, src_hbm, dst_hbm, ea_hbm, we_hbm,
               out_hbm, den_hbm,
               srcb, dstb, sdst, eab, qb, kb, vb, ob, exb, web, acc_sh, den_sh,
               sem1, sem2, sem3, ss):
    cid = lax.axis_index("c")
    sid = lax.axis_index("s")
    wid = cid * NS + sid
    base_edge = wid * EPW

    pltpu.sync_copy(we_hbm, web)

    z16 = jnp.zeros((16,), jnp.float32)

    def zero_ob_body(i, carry):
        ob[i // 8, pl.ds((i % 8) * 16, 16)] = z16
        return carry

    lax.fori_loop(0, C * 8, zero_ob_body, 0)

    def zero_ex_body(i, carry):
        exb[pl.ds(i * 16, 16)] = z16
        return carry

    lax.fori_loop(0, C // 16, zero_ex_body, 0)

    for i in range(RPT // C):
        pltpu.sync_copy(ob, acc_sh.at[pl.ds(sid * RPT + i * C, C)])
        pltpu.sync_copy(exb, den_sh.at[pl.ds(sid * RPT + i * C, C)])
    plsc.subcore_barrier()

    wevs = [web[pl.ds(16 * j, 16)] for j in range(8)]
    iota16 = lax.iota(jnp.int32, 16)

    for z in range(C // 16):
        sdst[pl.ds(z * 16, 16)] = iota16 * 0
    pltpu.async_copy(ob, acc_sh.at[sdst], ss, add=True)
    pltpu.async_copy(exb, den_sh.at[sdst], ss, add=True)

    def chunk_body(ci, carry):
        bo = base_edge + ci * C
        pltpu.sync_copy(src_hbm.at[pl.ds(bo, C)], srcb)
        pltpu.sync_copy(dst_hbm.at[pl.ds(bo, C)], dstb)
        pltpu.sync_copy(ea_hbm.at[pl.ds(bo, C)], eab)
        cp1 = pltpu.async_copy(q_hbm.at[dstb], qb, sem1)
        cp2 = pltpu.async_copy(k_hbm.at[srcb], kb, sem2)
        cp3 = pltpu.async_copy(v_hbm.at[srcb], vb, sem3)
        cp1.wait()
        cp2.wait()
        cp3.wait()
        pltpu.make_async_copy(ob, acc_sh.at[sdst], ss).wait()
        pltpu.make_async_copy(exb, den_sh.at[sdst], ss).wait()

        def group_body(g, carry2):
            sl16 = pl.ds(g * LANES, LANES)
            ea16 = eab[sl16]
            sdst[sl16] = dstb[sl16]
            exg = jnp.zeros((16,), jnp.float32)
            for l in range(LANES):
                e = g * LANES + l
                eae = ea16[l]
                acc = jnp.zeros((16,), jnp.float32)
                for j in range(8):
                    sl = pl.ds(16 * j, 16)
                    acc = acc + qb[e, sl] * (kb[e, sl] + eae * wevs[j])
                alpha = jnp.sum(acc)
                exv = jnp.exp(jnp.full((16,), alpha, jnp.float32))
                for j in range(8):
                    sl = pl.ds(16 * j, 16)
                    ob[e, sl] = exv * (vb[e, sl] + eae * wevs[j])
                exg = exg + exv * jnp.where(iota16 == l, 1.0, 0.0)
            exb[sl16] = exg
            return carry2

        lax.fori_loop(0, C // LANES, group_body, 0)
        pltpu.async_copy(ob, acc_sh.at[sdst], ss, add=True)
        pltpu.async_copy(exb, den_sh.at[sdst], ss, add=True)
        return carry

    lax.fori_loop(0, NCHUNK, chunk_body, 0)
    pltpu.make_async_copy(ob, acc_sh.at[sdst], ss).wait()
    pltpu.make_async_copy(exb, den_sh.at[sdst], ss).wait()
    plsc.subcore_barrier()

    pltpu.sync_copy(acc_sh.at[pl.ds(sid * RPT, RPT)],
                    out_hbm.at[cid].at[pl.ds(sid * RPT, RPT)])
    pltpu.sync_copy(den_sh.at[pl.ds(sid * RPT, RPT)],
                    den_hbm.at[cid].at[pl.ds(sid * RPT, RPT)])


_edge_call = functools.partial(
    pl.kernel,
    out_type=(jax.ShapeDtypeStruct((NC, NPAD, H), jnp.float32),
              jax.ShapeDtypeStruct((NC, NPAD), jnp.float32)),
    mesh=_mesh,
    compiler_params=pltpu.CompilerParams(needs_layout_passes=False,
                                         use_tc_tiling_on_sc=False),
    scratch_types=[
        pltpu.VMEM((C,), jnp.int32),
        pltpu.VMEM((C,), jnp.int32),
        pltpu.VMEM((C,), jnp.int32),
        pltpu.VMEM((C,), jnp.float32),
        pltpu.VMEM((C, H), jnp.float32),
        pltpu.VMEM((C, H), jnp.float32),
        pltpu.VMEM((C, H), jnp.float32),
        pltpu.VMEM((C, H), jnp.float32),
        pltpu.VMEM((C,), jnp.float32),
        pltpu.VMEM((H,), jnp.float32),
        pltpu.VMEM_SHARED((NPAD, H), jnp.float32),
        pltpu.VMEM_SHARED((NPAD,), jnp.float32),
        pltpu.SemaphoreType.DMA,
        pltpu.SemaphoreType.DMA,
        pltpu.SemaphoreType.DMA,
        pltpu.SemaphoreType.DMA,
    ],
)(_edge_body)


def _edge1_body(q_hbm, k_hbm, v_hbm, src_hbm, dst_hbm, ea_hbm, we_hbm,
                num_hbm, den_hbm, srcb, dstb, eab, qbuf, kbuf, vbuf,
                numb, exb, web, num_sh, den_sh):
    cid = lax.axis_index("c")
    sid = lax.axis_index("s")
    wid = cid * NS + sid

    pltpu.sync_copy(we_hbm, web)
    pltpu.sync_copy(q_hbm, qbuf)
    pltpu.sync_copy(k_hbm, kbuf)
    pltpu.sync_copy(v_hbm, vbuf)

    z16 = jnp.zeros((16,), jnp.float32)

    def zero_small(i, carry):
        numb[pl.ds(i * 16, 16)] = z16
        exb[pl.ds(i * 16, 16)] = z16
        return carry

    lax.fori_loop(0, C // 16, zero_small, 0)
    for i in range(RPT // C):
        pltpu.sync_copy(numb, num_sh.at[pl.ds(sid * RPT + i * C, C)])
        pltpu.sync_copy(exb, den_sh.at[pl.ds(sid * RPT + i * C, C)])
    plsc.subcore_barrier()

    base_edge = wid * EPW
    wev = web[...]

    def chunk_body(ci, carry):
        b = base_edge + ci * C
        pltpu.sync_copy(src_hbm.at[pl.ds(b, C)], srcb)
        pltpu.sync_copy(dst_hbm.at[pl.ds(b, C)], dstb)
        pltpu.sync_copy(ea_hbm.at[pl.ds(b, C)], eab)

        def group_body(g, carry2):
            sl = pl.ds(g * LANES, LANES)
            srcv = srcb[sl]
            dstv = dstb[sl]
            eav = eab[sl]
            k16 = plsc.load_gather(kbuf, [srcv])
            q16 = plsc.load_gather(qbuf, [dstv])
            v16 = plsc.load_gather(vbuf, [srcv])
            ew = eav * wev
            ex = jnp.exp(q16 * (k16 + ew))
            numb[sl] = ex * (v16 + ew)
            exb[sl] = ex
            return carry2

        lax.fori_loop(0, C // LANES, group_body, 0)
        pltpu.sync_copy(numb, num_sh.at[dstb], add=True)
        pltpu.sync_copy(exb, den_sh.at[dstb], add=True)
        return carry

    lax.fori_loop(0, NCHUNK, chunk_body, 0)
    plsc.subcore_barrier()

    pltpu.sync_copy(num_sh.at[pl.ds(sid * RPT, RPT)],
                    num_hbm.at[cid].at[pl.ds(sid * RPT, RPT)])
    pltpu.sync_copy(den_sh.at[pl.ds(sid * RPT, RPT)],
                    den_hbm.at[cid].at[pl.ds(sid * RPT, RPT)])


_edge1_call = functools.partial(
    pl.kernel,
    out_type=(jax.ShapeDtypeStruct((NC, NPAD), jnp.float32),
              jax.ShapeDtypeStruct((NC, NPAD), jnp.float32)),
    mesh=_mesh,
    compiler_params=pltpu.CompilerParams(needs_layout_passes=False,
                                         use_tc_tiling_on_sc=False),
    scratch_types=[
        pltpu.VMEM((C,), jnp.int32),
        pltpu.VMEM((C,), jnp.int32),
        pltpu.VMEM((C,), jnp.float32),
        pltpu.VMEM((NPAD,), jnp.float32),
        pltpu.VMEM((NPAD,), jnp.float32),
        pltpu.VMEM((NPAD,), jnp.float32),
        pltpu.VMEM((C,), jnp.float32),
        pltpu.VMEM((C,), jnp.float32),
        pltpu.VMEM((16,), jnp.float32),
        pltpu.VMEM_SHARED((NPAD,), jnp.float32),
        pltpu.VMEM_SHARED((NPAD,), jnp.float32),
    ],
)(_edge1_body)


def _combine_body(acc_ref, den_ref, skip_ref, out_ref):
    num = (acc_ref[0] + acc_ref[1])[:N]
    den = (den_ref[0] + den_ref[1])[:N]
    h = num / (den[:, None] + 1e-16) + skip_ref[...]
    out_ref[...] = jnp.where(h >= 0.0, h, 0.01 * h)


def _combine(acc, den, skip):
    return pl.pallas_call(
        _combine_body,
        out_shape=jax.ShapeDtypeStruct((N, H), jnp.float32),
    )(acc, den, skip)


def _mean_proj_body(a_ref, b_ref, c_ref, d_ref, w_ref, bias_ref, y_ref):
    hm = 0.25 * (a_ref[...] + b_ref[...] + c_ref[...] + d_ref[...])
    y_ref[...] = jnp.dot(hm, w_ref[...],
                         preferred_element_type=jnp.float32) + bias_ref[...]


def _final_body(num_ref, den_ref, skip_ref, out_ref):
    num = (num_ref[0] + num_ref[1])[:N]
    den = (den_ref[0] + den_ref[1])[:N]
    out_ref[...] = (num / (den + 1e-16))[:, None] + skip_ref[...]


def kernel(x_seq, edge_index_seq, edge_attr_seq, gru, conv1, conv2, conv_out):
    gru_out = _gru_forward(x_seq, gru)
    edge_index_seq = edge_index_seq.astype(jnp.int32)

    h_list = []
    for t in range(T):
        h_t = gru_out[t]
        srcf = edge_index_seq[t, 0]
        dstf = edge_index_seq[t, 1]
        eaf = edge_attr_seq[t, :, 0]
        for p in (conv1, conv2):
            q, k, v, skip = _projections(h_t, p)
            acc, den = _edge_call(q, k, v, srcf, dstf, eaf, p['We'][0])
            h_t = _combine(acc, den, skip)
        h_list.append(h_t)

    p = conv_out
    w4 = jnp.concatenate([p['Wq'], p['Wk'], p['Wv'], p['Wskip']], axis=1)
    w4 = jnp.pad(w4, ((0, 0), (0, 4)))
    b4 = jnp.pad(jnp.concatenate([p['bq'], p['bk'], p['bv'], p['bskip']]),
                 (0, 4)).reshape(1, 8)
    y = pl.pallas_call(
        _mean_proj_body,
        out_shape=jax.ShapeDtypeStruct((N, 8), jnp.float32),
    )(*h_list, w4, b4)

    src = edge_index_seq[T - 1, 0]
    dst = edge_index_seq[T - 1, 1]
    ea = edge_attr_seq[T - 1, :, 0]
    qv = jnp.pad(y[:, 0], (0, NPAD - N))
    kv = jnp.pad(y[:, 1], (0, NPAD - N))
    vv = jnp.pad(y[:, 2], (0, NPAD - N))
    skip = y[:, 3:4]
    we16 = jnp.full((16,), p['We'][0, 0], jnp.float32)
    num, den = _edge1_call(qv, kv, vv, src, dst, ea, we16)
    out = pl.pallas_call(
        _final_body,
        out_shape=jax.ShapeDtypeStruct((N, 1), jnp.float32),
    )(num, den, skip)
    return out


# speedup vs baseline: 2.1167x; 1.1761x over previous
import functools
import math
import jax
import jax.numpy as jnp
from jax import lax
from jax.experimental import pallas as pl
from jax.experimental.pallas import tpu as pltpu
from jax.experimental.pallas import tpu_sc as plsc

T, N, E, H = 4, 10000, 320000, 128
INV_SQRT_D = 1.0 / math.sqrt(128.0)

NC, NS, LANES = 2, 16, 16
NW = NC * NS
EPW = E // NW
C = 80
NCHUNK = EPW // C
NPAD = 10240
RPT = NPAD // NS

_mesh = plsc.VectorSubcoreMesh(
    core_axis_name="c", subcore_axis_name="s", num_cores=NC, num_subcores=NS)


def _gru_body(x_ref, wih_ref, whh_ref, bih_ref, bhh_ref, out_ref, h_ref):
    t = pl.program_id(0)

    @pl.when(t == 0)
    def _():
        h_ref[...] = jnp.zeros_like(h_ref)

    x = x_ref[0]
    h = h_ref[...]
    gi = lax.dot_general(x, wih_ref[...], (((1,), (1,)), ((), ())),
                         preferred_element_type=jnp.float32) + bih_ref[...]
    gh = lax.dot_general(h, whh_ref[...], (((1,), (1,)), ((), ())),
                         preferred_element_type=jnp.float32) + bhh_ref[...]
    r = jax.nn.sigmoid(gi[:, :H] + gh[:, :H])
    z = jax.nn.sigmoid(gi[:, H:2 * H] + gh[:, H:2 * H])
    n = jnp.tanh(gi[:, 2 * H:] + r * gh[:, 2 * H:])
    h = (1.0 - z) * n + z * h
    h_ref[...] = h
    out_ref[0] = h


def _gru_forward(x_seq, gru):
    bih = gru['b_ih'].reshape(1, 3 * H)
    bhh = gru['b_hh'].reshape(1, 3 * H)
    return pl.pallas_call(
        _gru_body,
        grid=(T,),
        in_specs=[
            pl.BlockSpec((1, N, H), lambda t: (t, 0, 0)),
            pl.BlockSpec((3 * H, H), lambda t: (0, 0)),
            pl.BlockSpec((3 * H, H), lambda t: (0, 0)),
            pl.BlockSpec((1, 3 * H), lambda t: (0, 0)),
            pl.BlockSpec((1, 3 * H), lambda t: (0, 0)),
        ],
        out_specs=pl.BlockSpec((1, N, H), lambda t: (t, 0, 0)),
        out_shape=jax.ShapeDtypeStruct((T, N, H), jnp.float32),
        scratch_shapes=[pltpu.VMEM((N, H), jnp.float32)],
    )(x_seq, gru['W_ih'], gru['W_hh'], bih, bhh)


def _proj_body(h_ref, w_ref, b_ref, q_ref, k_ref, v_ref, s_ref):
    y = jnp.dot(h_ref[...], w_ref[...],
                preferred_element_type=jnp.float32) + b_ref[...]
    q_ref[...] = y[:, :H]
    k_ref[...] = y[:, H:2 * H]
    v_ref[...] = y[:, 2 * H:3 * H]
    s_ref[...] = y[:, 3 * H:]


def _projections(h, p):
    w = jnp.concatenate([p['Wq'] * INV_SQRT_D, p['Wk'], p['Wv'], p['Wskip']], axis=1)
    b = jnp.concatenate([p['bq'] * INV_SQRT_D, p['bk'], p['bv'], p['bskip']]).reshape(1, 4 * H)
    shp = jax.ShapeDtypeStruct((N, H), jnp.float32)
    return pl.pallas_call(
        _proj_body,
        out_shape=(shp, shp, shp, shp),
    )(h, w, b)


def _edge_body(q_hbm, k_hbm, v_hbm, src_hbm, dst_hbm, ea_hbm, we_hbm,
               out_hbm, den_hbm,
               srcb, dstb, sdst, eab, qb, kb, vb, ob, exb, web, acc_sh, den_sh,
               sem1, sem2, sem3, ss, si):
    cid = lax.axis_index("c")
    sid = lax.axis_index("s")
    wid = cid * NS + sid
    base_edge = wid * EPW

    pltpu.sync_copy(we_hbm, web)

    z16 = jnp.zeros((16,), jnp.float32)

    def zero_ob_body(i, carry):
        ob[i // 8, pl.ds((i % 8) * 16, 16)] = z16
        return carry

    lax.fori_loop(0, C * 8, zero_ob_body, 0)

    def zero_ex_body(i, carry):
        exb[pl.ds(i * 16, 16)] = z16
        return carry

    lax.fori_loop(0, C // 16, zero_ex_body, 0)

    for i in range(RPT // C):
        pltpu.sync_copy(ob, acc_sh.at[pl.ds(sid * RPT + i * C, C)])
        pltpu.sync_copy(exb, den_sh.at[pl.ds(sid * RPT + i * C, C)])
    plsc.subcore_barrier()

    wevs = [web[pl.ds(16 * j, 16)] for j in range(8)]
    iota16 = lax.iota(jnp.int32, 16)

    for z in range(C // 16):
        sdst[pl.ds(z * 16, 16)] = iota16 * 0
    pltpu.async_copy(ob, acc_sh.at[sdst], ss, add=True)
    pltpu.async_copy(exb, den_sh.at[sdst], ss, add=True)

    def chunk_body(ci, carry):
        bo = base_edge + ci * C
        ca = pltpu.async_copy(src_hbm.at[pl.ds(bo, C)], srcb, si)
        cb = pltpu.async_copy(dst_hbm.at[pl.ds(bo, C)], dstb, si)
        cc = pltpu.async_copy(ea_hbm.at[pl.ds(bo, C)], eab, si)
        ca.wait()
        cb.wait()
        cc.wait()
        cp1 = pltpu.async_copy(q_hbm.at[dstb], qb, sem1)
        cp2 = pltpu.async_copy(k_hbm.at[srcb], kb, sem2)
        cp3 = pltpu.async_copy(v_hbm.at[srcb], vb, sem3)
        cp1.wait()
        cp2.wait()
        cp3.wait()
        pltpu.make_async_copy(ob, acc_sh.at[sdst], ss).wait()
        pltpu.make_async_copy(exb, den_sh.at[sdst], ss).wait()

        def group_body(g, carry2):
            sl16 = pl.ds(g * LANES, LANES)
            ea16 = eab[sl16]
            sdst[sl16] = dstb[sl16]
            exg = jnp.zeros((16,), jnp.float32)
            for l in range(LANES):
                e = g * LANES + l
                eae = ea16[l]
                acc = jnp.zeros((16,), jnp.float32)
                for j in range(8):
                    sl = pl.ds(16 * j, 16)
                    acc = acc + qb[e, sl] * (kb[e, sl] + eae * wevs[j])
                alpha = jnp.sum(acc)
                exv = jnp.exp(jnp.full((16,), alpha, jnp.float32))
                for j in range(8):
                    sl = pl.ds(16 * j, 16)
                    ob[e, sl] = exv * (vb[e, sl] + eae * wevs[j])
                exg = exg + exv * jnp.where(iota16 == l, 1.0, 0.0)
            exb[sl16] = exg
            return carry2

        lax.fori_loop(0, C // LANES, group_body, 0)
        pltpu.async_copy(ob, acc_sh.at[sdst], ss, add=True)
        pltpu.async_copy(exb, den_sh.at[sdst], ss, add=True)
        return carry

    lax.fori_loop(0, NCHUNK, chunk_body, 0)
    pltpu.make_async_copy(ob, acc_sh.at[sdst], ss).wait()
    pltpu.make_async_copy(exb, den_sh.at[sdst], ss).wait()
    plsc.subcore_barrier()

    pltpu.sync_copy(acc_sh.at[pl.ds(sid * RPT, RPT)],
                    out_hbm.at[cid].at[pl.ds(sid * RPT, RPT)])
    pltpu.sync_copy(den_sh.at[pl.ds(sid * RPT, RPT)],
                    den_hbm.at[cid].at[pl.ds(sid * RPT, RPT)])


_edge_call = functools.partial(
    pl.kernel,
    out_type=(jax.ShapeDtypeStruct((NC, NPAD, H), jnp.float32),
              jax.ShapeDtypeStruct((NC, NPAD), jnp.float32)),
    mesh=_mesh,
    compiler_params=pltpu.CompilerParams(needs_layout_passes=False,
                                         use_tc_tiling_on_sc=False),
    scratch_types=[
        pltpu.VMEM((C,), jnp.int32),
        pltpu.VMEM((C,), jnp.int32),
        pltpu.VMEM((C,), jnp.int32),
        pltpu.VMEM((C,), jnp.float32),
        pltpu.VMEM((C, H), jnp.float32),
        pltpu.VMEM((C, H), jnp.float32),
        pltpu.VMEM((C, H), jnp.float32),
        pltpu.VMEM((C, H), jnp.float32),
        pltpu.VMEM((C,), jnp.float32),
        pltpu.VMEM((H,), jnp.float32),
        pltpu.VMEM_SHARED((NPAD, H), jnp.float32),
        pltpu.VMEM_SHARED((NPAD,), jnp.float32),
        pltpu.SemaphoreType.DMA,
        pltpu.SemaphoreType.DMA,
        pltpu.SemaphoreType.DMA,
        pltpu.SemaphoreType.DMA,
        pltpu.SemaphoreType.DMA,
    ],
)(_edge_body)


def _edge1_body(q_hbm, k_hbm, v_hbm, src_hbm, dst_hbm, ea_hbm, we_hbm,
                num_hbm, den_hbm, srcb, dstb, eab, qbuf, kbuf, vbuf,
                numb, exb, web, num_sh, den_sh):
    cid = lax.axis_index("c")
    sid = lax.axis_index("s")
    wid = cid * NS + sid

    pltpu.sync_copy(we_hbm, web)
    pltpu.sync_copy(q_hbm, qbuf)
    pltpu.sync_copy(k_hbm, kbuf)
    pltpu.sync_copy(v_hbm, vbuf)

    z16 = jnp.zeros((16,), jnp.float32)

    def zero_small(i, carry):
        numb[pl.ds(i * 16, 16)] = z16
        exb[pl.ds(i * 16, 16)] = z16
        return carry

    lax.fori_loop(0, C // 16, zero_small, 0)
    for i in range(RPT // C):
        pltpu.sync_copy(numb, num_sh.at[pl.ds(sid * RPT + i * C, C)])
        pltpu.sync_copy(exb, den_sh.at[pl.ds(sid * RPT + i * C, C)])
    plsc.subcore_barrier()

    base_edge = wid * EPW
    wev = web[...]

    def chunk_body(ci, carry):
        b = base_edge + ci * C
        pltpu.sync_copy(src_hbm.at[pl.ds(b, C)], srcb)
        pltpu.sync_copy(dst_hbm.at[pl.ds(b, C)], dstb)
        pltpu.sync_copy(ea_hbm.at[pl.ds(b, C)], eab)

        def group_body(g, carry2):
            sl = pl.ds(g * LANES, LANES)
            srcv = srcb[sl]
            dstv = dstb[sl]
            eav = eab[sl]
            k16 = plsc.load_gather(kbuf, [srcv])
            q16 = plsc.load_gather(qbuf, [dstv])
            v16 = plsc.load_gather(vbuf, [srcv])
            ew = eav * wev
            ex = jnp.exp(q16 * (k16 + ew))
            numb[sl] = ex * (v16 + ew)
            exb[sl] = ex
            return carry2

        lax.fori_loop(0, C // LANES, group_body, 0)
        pltpu.sync_copy(numb, num_sh.at[dstb], add=True)
        pltpu.sync_copy(exb, den_sh.at[dstb], add=True)
        return carry

    lax.fori_loop(0, NCHUNK, chunk_body, 0)
    plsc.subcore_barrier()

    pltpu.sync_copy(num_sh.at[pl.ds(sid * RPT, RPT)],
                    num_hbm.at[cid].at[pl.ds(sid * RPT, RPT)])
    pltpu.sync_copy(den_sh.at[pl.ds(sid * RPT, RPT)],
                    den_hbm.at[cid].at[pl.ds(sid * RPT, RPT)])


_edge1_call = functools.partial(
    pl.kernel,
    out_type=(jax.ShapeDtypeStruct((NC, NPAD), jnp.float32),
              jax.ShapeDtypeStruct((NC, NPAD), jnp.float32)),
    mesh=_mesh,
    compiler_params=pltpu.CompilerParams(needs_layout_passes=False,
                                         use_tc_tiling_on_sc=False),
    scratch_types=[
        pltpu.VMEM((C,), jnp.int32),
        pltpu.VMEM((C,), jnp.int32),
        pltpu.VMEM((C,), jnp.float32),
        pltpu.VMEM((NPAD,), jnp.float32),
        pltpu.VMEM((NPAD,), jnp.float32),
        pltpu.VMEM((NPAD,), jnp.float32),
        pltpu.VMEM((C,), jnp.float32),
        pltpu.VMEM((C,), jnp.float32),
        pltpu.VMEM((16,), jnp.float32),
        pltpu.VMEM_SHARED((NPAD,), jnp.float32),
        pltpu.VMEM_SHARED((NPAD,), jnp.float32),
    ],
)(_edge1_body)


def _combine_body(acc_ref, den_ref, skip_ref, out_ref):
    num = (acc_ref[0] + acc_ref[1])[:N]
    den = (den_ref[0] + den_ref[1])[:N]
    h = num / (den[:, None] + 1e-16) + skip_ref[...]
    out_ref[...] = jnp.where(h >= 0.0, h, 0.01 * h)


def _combine(acc, den, skip):
    return pl.pallas_call(
        _combine_body,
        out_shape=jax.ShapeDtypeStruct((N, H), jnp.float32),
    )(acc, den, skip)


def _mean_proj_body(a_ref, b_ref, c_ref, d_ref, w_ref, bias_ref, y_ref):
    hm = 0.25 * (a_ref[...] + b_ref[...] + c_ref[...] + d_ref[...])
    y_ref[...] = jnp.dot(hm, w_ref[...],
                         preferred_element_type=jnp.float32) + bias_ref[...]


def _final_body(num_ref, den_ref, skip_ref, out_ref):
    num = (num_ref[0] + num_ref[1])[:N]
    den = (den_ref[0] + den_ref[1])[:N]
    out_ref[...] = (num / (den + 1e-16))[:, None] + skip_ref[...]


def kernel(x_seq, edge_index_seq, edge_attr_seq, gru, conv1, conv2, conv_out):
    gru_out = _gru_forward(x_seq, gru)
    edge_index_seq = edge_index_seq.astype(jnp.int32)

    h_list = []
    for t in range(T):
        h_t = gru_out[t]
        srcf = edge_index_seq[t, 0]
        dstf = edge_index_seq[t, 1]
        eaf = edge_attr_seq[t, :, 0]
        for p in (conv1, conv2):
            q, k, v, skip = _projections(h_t, p)
            acc, den = _edge_call(q, k, v, srcf, dstf, eaf, p['We'][0])
            h_t = _combine(acc, den, skip)
        h_list.append(h_t)

    p = conv_out
    w4 = jnp.concatenate([p['Wq'], p['Wk'], p['Wv'], p['Wskip']], axis=1)
    w4 = jnp.pad(w4, ((0, 0), (0, 4)))
    b4 = jnp.pad(jnp.concatenate([p['bq'], p['bk'], p['bv'], p['bskip']]),
                 (0, 4)).reshape(1, 8)
    y = pl.pallas_call(
        _mean_proj_body,
        out_shape=jax.ShapeDtypeStruct((N, 8), jnp.float32),
    )(*h_list, w4, b4)

    src = edge_index_seq[T - 1, 0]
    dst = edge_index_seq[T - 1, 1]
    ea = edge_attr_seq[T - 1, :, 0]
    qv = jnp.pad(y[:, 0], (0, NPAD - N))
    kv = jnp.pad(y[:, 1], (0, NPAD - N))
    vv = jnp.pad(y[:, 2], (0, NPAD - N))
    skip = y[:, 3:4]
    we16 = jnp.full((16,), p['We'][0, 0], jnp.float32)
    num, den = _edge1_call(qv, kv, vv, src, dst, ea, we16)
    out = pl.pallas_call(
        _final_body,
        out_shape=jax.ShapeDtypeStruct((N, 1), jnp.float32),
    )(num, den, skip)
    return out
